# Initial kernel scaffold; baseline (speedup 1.0000x reference)
import jax
import jax.numpy as jnp
from jax.experimental import pallas as pl


def kernel(h, src, dst, W0, b0, W1, b1, W2, b2):
    raise NotImplementedError("write your pallas kernel here")


# speedup vs baseline: 10.9898x; 10.9898x over previous
import functools

import jax
import jax.numpy as jnp
from jax import lax
from jax.experimental import pallas as pl
from jax.experimental.pallas import tpu as pltpu
from jax.experimental.pallas import tpu_sc as plsc

N_NODES = 10000
NP = 10240
D = 128
DEGW = 16
NC, NS = 2, 16
NW = NC * NS
E2 = 2 * 320000
E2P = 655360
EPW = E2P // NW
CH = 128
NCH = EPW // CH
IB = 32
RPZ = NP // NS

_mesh = plsc.VectorSubcoreMesh(core_axis_name="c", subcore_axis_name="s")


def _sc_degree(d2, zerosD, onesD):
    @functools.partial(
        pl.kernel,
        out_type=jax.ShapeDtypeStruct((NC, NP, D), jnp.float32),
        mesh=_mesh,
        scratch_types=[
            pltpu.VMEM((IB, CH), jnp.int32),
            pltpu.VMEM((CH, D), jnp.float32),
            pltpu.VMEM_SHARED((NP, D), jnp.float32),
        ],
    )
    def k(d_hbm, z_hbm, o_hbm, out_hbm, didx, ones, acc):
        cid = lax.axis_index("c")
        sid = lax.axis_index("s")
        wid = sid * NC + cid
        pltpu.sync_copy(z_hbm.at[pl.ds(sid * RPZ, RPZ)],
                        acc.at[pl.ds(sid * RPZ, RPZ)])
        pltpu.sync_copy(o_hbm, ones)
        plsc.subcore_barrier()

        @pl.loop(0, NCH // IB)
        def _(blk):
            pltpu.sync_copy(d_hbm.at[pl.ds(wid * NCH + blk * IB, IB)], didx)

            @pl.loop(0, IB)
            def _(c):
                pltpu.sync_copy(ones, acc.at[didx.at[c]], add=True)

        plsc.subcore_barrier()
        pltpu.sync_copy(acc.at[pl.ds(sid * RPZ, RPZ)],
                        out_hbm.at[cid, pl.ds(sid * RPZ, RPZ)])

    return k(d2, zerosD, onesD)


def _sc_scatter(m, s2, d2, zerosD):
    @functools.partial(
        pl.kernel,
        out_type=jax.ShapeDtypeStruct((NC, NP, D), jnp.float32),
        mesh=_mesh,
        scratch_types=[
            pltpu.VMEM((IB, CH), jnp.int32),
            pltpu.VMEM((IB, CH), jnp.int32),
            pltpu.VMEM((CH, D), jnp.float32),
            pltpu.VMEM_SHARED((NP, D), jnp.float32),
            pltpu.SemaphoreType.DMA,
        ],
    )
    def k(m_hbm, s_hbm, d_hbm, z_hbm, out_hbm, sidx, didx, rows, acc, sem):
        cid = lax.axis_index("c")
        sid = lax.axis_index("s")
        wid = sid * NC + cid
        pltpu.sync_copy(z_hbm.at[pl.ds(sid * RPZ, RPZ)],
                        acc.at[pl.ds(sid * RPZ, RPZ)])
        plsc.subcore_barrier()

        @pl.loop(0, NCH // IB)
        def _(blk):
            base = wid * NCH + blk * IB
            pltpu.sync_copy(s_hbm.at[pl.ds(base, IB)], sidx)
            pltpu.sync_copy(d_hbm.at[pl.ds(base, IB)], didx)

            @pl.loop(0, IB)
            def _(c):
                pltpu.async_copy(m_hbm.at[sidx.at[c]], rows, sem).wait()
                pltpu.sync_copy(rows, acc.at[didx.at[c]], add=True)

        plsc.subcore_barrier()
        pltpu.sync_copy(acc.at[pl.ds(sid * RPZ, RPZ)],
                        out_hbm.at[cid, pl.ds(sid * RPZ, RPZ)])

    return k(m, s2, d2, zerosD)


def _tc_norm_m(degp, h_pad):
    BR = 512

    def body(deg_ref, h_ref, norm_ref, m_ref):
        p = deg_ref[...]
        deg = p[0, :, 0:1] + p[1, :, 0:1] + 1.0
        nb = jnp.broadcast_to(lax.rsqrt(deg), (BR, D))
        norm_ref[...] = nb
        m_ref[...] = h_ref[...] * nb

    return pl.pallas_call(
        body,
        grid=(NP // BR,),
        in_specs=[
            pl.BlockSpec((NC, BR, D), lambda i: (0, i, 0)),
            pl.BlockSpec((BR, D), lambda i: (i, 0)),
        ],
        out_specs=[
            pl.BlockSpec((BR, D), lambda i: (i, 0)),
            pl.BlockSpec((BR, D), lambda i: (i, 0)),
        ],
        out_shape=[
            jax.ShapeDtypeStruct((NP, D), jnp.float32),
            jax.ShapeDtypeStruct((NP, D), jnp.float32),
        ],
    )(degp, h_pad)


def _tc_layer(p, m, normb, W, b2d, last):
    rows = N_NODES if last else NP
    BR = 1000 if last else 512

    def body(p_ref, m_ref, n_ref, w_ref, b_ref, o_ref):
        pp = p_ref[...]
        agg = (pp[0] + pp[1] + m_ref[...]) * n_ref[...]
        y = jnp.dot(agg, w_ref[...], preferred_element_type=jnp.float32)
        y = y + b_ref[...]
        if not last:
            y = y * n_ref[...]
        o_ref[...] = y

    return pl.pallas_call(
        body,
        grid=(rows // BR,),
        in_specs=[
            pl.BlockSpec((NC, BR, D), lambda i: (0, i, 0)),
            pl.BlockSpec((BR, D), lambda i: (i, 0)),
            pl.BlockSpec((BR, D), lambda i: (i, 0)),
            pl.BlockSpec((D, D), lambda i: (0, 0)),
            pl.BlockSpec((1, D), lambda i: (0, 0)),
        ],
        out_specs=pl.BlockSpec((BR, D), lambda i: (i, 0)),
        out_shape=jax.ShapeDtypeStruct((rows, D), jnp.float32),
    )(p, m, normb, W, b2d)


def kernel(h, src, dst, W0, b0, W1, b1, W2, b2):
    pad = E2P - E2
    pad_idx = (jnp.arange(pad, dtype=jnp.int32) % (NP - N_NODES)) + N_NODES
    s2 = jnp.concatenate([src, dst, pad_idx]).reshape(E2P // CH, CH)
    d2 = jnp.concatenate([dst, src, pad_idx]).reshape(E2P // CH, CH)
    h_pad = jnp.pad(h, ((0, NP - N_NODES), (0, 0)))
    zerosD = jnp.zeros((NP, D), jnp.float32)
    onesD = jnp.ones((CH, D), jnp.float32)

    degp = _sc_degree(d2, zerosD, onesD)
    normb, m = _tc_norm_m(degp, h_pad)
    out = None
    for i, (W, b) in enumerate(((W0, b0), (W1, b1), (W2, b2))):
        p = _sc_scatter(m, s2, d2, zerosD)
        out = _tc_layer(p, m, normb, W, b.reshape(1, D), last=(i == 2))
        m = out
    return out


# speedup vs baseline: 12.9931x; 1.1823x over previous
import functools

import jax
import jax.numpy as jnp
from jax import lax
from jax.experimental import pallas as pl
from jax.experimental.pallas import tpu as pltpu
from jax.experimental.pallas import tpu_sc as plsc

N_NODES = 10000
NP = 10240
D = 128
DEGW = 16
NC, NS = 2, 16
NW = NC * NS
E2 = 2 * 320000
E2P = 655360
EPW = E2P // NW
CH = 128
NCH = EPW // CH
IB = 32
RPZ = NP // NS

_mesh = plsc.VectorSubcoreMesh(core_axis_name="c", subcore_axis_name="s")


def _sc_degree(d2, zerosD, onesD):
    @functools.partial(
        pl.kernel,
        out_type=jax.ShapeDtypeStruct((NC, NP, D), jnp.float32),
        mesh=_mesh,
        scratch_types=[
            pltpu.VMEM((IB, CH), jnp.int32),
            pltpu.VMEM((CH, D), jnp.float32),
            pltpu.VMEM_SHARED((NP, D), jnp.float32),
            pltpu.SemaphoreType.DMA,
        ],
    )
    def k(d_hbm, z_hbm, o_hbm, out_hbm, didx, ones, acc, sem):
        cid = lax.axis_index("c")
        sid = lax.axis_index("s")
        wid = sid * NC + cid
        pltpu.sync_copy(z_hbm.at[pl.ds(sid * RPZ, RPZ)],
                        acc.at[pl.ds(sid * RPZ, RPZ)])
        pltpu.sync_copy(o_hbm, ones)
        plsc.subcore_barrier()

        @pl.loop(0, NCH // IB)
        def _(blk):
            pltpu.sync_copy(d_hbm.at[pl.ds(wid * NCH + blk * IB, IB)], didx)

            @pl.loop(0, IB)
            def _(c):
                pltpu.async_copy(ones, acc.at[didx.at[c]], sem, add=True)

            @pl.loop(0, IB)
            def _(c):
                pltpu.make_async_copy(ones, acc.at[didx.at[0]], sem).wait()

        plsc.subcore_barrier()
        pltpu.sync_copy(acc.at[pl.ds(sid * RPZ, RPZ)],
                        out_hbm.at[cid, pl.ds(sid * RPZ, RPZ)])

    return k(d2, zerosD, onesD)


def _sc_scatter(m, s2, d2, zerosD):
    @functools.partial(
        pl.kernel,
        out_type=jax.ShapeDtypeStruct((NC, NP, D), jnp.float32),
        mesh=_mesh,
        scratch_types=[
            pltpu.VMEM((IB, CH), jnp.int32),
            pltpu.VMEM((IB, CH), jnp.int32),
            pltpu.VMEM((CH, D), jnp.float32),
            pltpu.VMEM((CH, D), jnp.float32),
            pltpu.VMEM_SHARED((NP, D), jnp.float32),
            pltpu.SemaphoreType.DMA,
            pltpu.SemaphoreType.DMA,
            pltpu.SemaphoreType.DMA,
            pltpu.SemaphoreType.DMA,
        ],
    )
    def k(m_hbm, s_hbm, d_hbm, z_hbm, out_hbm, sidx, didx, b0, b1, acc,
          g0, g1, s0, s1):
        cid = lax.axis_index("c")
        sid = lax.axis_index("s")
        wid = sid * NC + cid
        pltpu.sync_copy(z_hbm.at[pl.ds(sid * RPZ, RPZ)],
                        acc.at[pl.ds(sid * RPZ, RPZ)])
        plsc.subcore_barrier()

        def gather(c, buf, sem):
            pltpu.async_copy(m_hbm.at[sidx.at[c]], buf, sem)

        def gwait(buf, sem):
            pltpu.make_async_copy(m_hbm.at[sidx.at[0]], buf, sem).wait()

        def scat(c, buf, sem):
            pltpu.async_copy(buf, acc.at[didx.at[c]], sem, add=True)

        def swait(buf, sem):
            pltpu.make_async_copy(buf, acc.at[didx.at[0]], sem).wait()

        @pl.loop(0, NCH // IB)
        def _(blk):
            base = wid * NCH + blk * IB
            pltpu.sync_copy(s_hbm.at[pl.ds(base, IB)], sidx)
            pltpu.sync_copy(d_hbm.at[pl.ds(base, IB)], didx)
            gather(0, b0, g0)
            gather(1, b1, g1)

            @pl.loop(0, IB // 2 - 1)
            def _(j):
                c = 2 * j
                gwait(b0, g0)
                scat(c, b0, s0)
                gwait(b1, g1)
                scat(c + 1, b1, s1)
                swait(b0, s0)
                gather(c + 2, b0, g0)
                swait(b1, s1)
                gather(c + 3, b1, g1)

            gwait(b0, g0)
            scat(IB - 2, b0, s0)
            gwait(b1, g1)
            scat(IB - 1, b1, s1)
            swait(b0, s0)
            swait(b1, s1)

        plsc.subcore_barrier()
        pltpu.sync_copy(acc.at[pl.ds(sid * RPZ, RPZ)],
                        out_hbm.at[cid, pl.ds(sid * RPZ, RPZ)])

    return k(m, s2, d2, zerosD)


def _tc_norm_m(degp, h_pad):
    BR = 512

    def body(deg_ref, h_ref, norm_ref, m_ref):
        p = deg_ref[...]
        deg = p[0, :, 0:1] + p[1, :, 0:1] + 1.0
        nb = jnp.broadcast_to(lax.rsqrt(deg), (BR, D))
        norm_ref[...] = nb
        m_ref[...] = h_ref[...] * nb

    return pl.pallas_call(
        body,
        grid=(NP // BR,),
        in_specs=[
            pl.BlockSpec((NC, BR, D), lambda i: (0, i, 0)),
            pl.BlockSpec((BR, D), lambda i: (i, 0)),
        ],
        out_specs=[
            pl.BlockSpec((BR, D), lambda i: (i, 0)),
            pl.BlockSpec((BR, D), lambda i: (i, 0)),
        ],
        out_shape=[
            jax.ShapeDtypeStruct((NP, D), jnp.float32),
            jax.ShapeDtypeStruct((NP, D), jnp.float32),
        ],
    )(degp, h_pad)


def _tc_layer(p, m, normb, W, b2d, last):
    rows = N_NODES if last else NP
    BR = 1000 if last else 512

    def body(p_ref, m_ref, n_ref, w_ref, b_ref, o_ref):
        pp = p_ref[...]
        agg = (pp[0] + pp[1] + m_ref[...]) * n_ref[...]
        y = jnp.dot(agg, w_ref[...], preferred_element_type=jnp.float32)
        y = y + b_ref[...]
        if not last:
            y = y * n_ref[...]
        o_ref[...] = y

    return pl.pallas_call(
        body,
        grid=(rows // BR,),
        in_specs=[
            pl.BlockSpec((NC, BR, D), lambda i: (0, i, 0)),
            pl.BlockSpec((BR, D), lambda i: (i, 0)),
            pl.BlockSpec((BR, D), lambda i: (i, 0)),
            pl.BlockSpec((D, D), lambda i: (0, 0)),
            pl.BlockSpec((1, D), lambda i: (0, 0)),
        ],
        out_specs=pl.BlockSpec((BR, D), lambda i: (i, 0)),
        out_shape=jax.ShapeDtypeStruct((rows, D), jnp.float32),
    )(p, m, normb, W, b2d)


def kernel(h, src, dst, W0, b0, W1, b1, W2, b2):
    pad = E2P - E2
    pad_idx = (jnp.arange(pad, dtype=jnp.int32) % (NP - N_NODES)) + N_NODES
    s2 = jnp.concatenate([src, dst, pad_idx]).reshape(E2P // CH, CH)
    d2 = jnp.concatenate([dst, src, pad_idx]).reshape(E2P // CH, CH)
    h_pad = jnp.pad(h, ((0, NP - N_NODES), (0, 0)))
    zerosD = jnp.zeros((NP, D), jnp.float32)
    onesD = jnp.ones((CH, D), jnp.float32)

    degp = _sc_degree(d2, zerosD, onesD)
    normb, m = _tc_norm_m(degp, h_pad)
    out = None
    for i, (W, b) in enumerate(((W0, b0), (W1, b1), (W2, b2))):
        p = _sc_scatter(m, s2, d2, zerosD)
        out = _tc_layer(p, m, normb, W, b.reshape(1, D), last=(i == 2))
        m = out
    return out


# speedup vs baseline: 15.8289x; 1.2183x over previous
import functools

import jax
import jax.numpy as jnp
from jax import lax
from jax.experimental import pallas as pl
from jax.experimental.pallas import tpu as pltpu
from jax.experimental.pallas import tpu_sc as plsc

N_NODES = 10000
NP = 10240
D = 128
DEGW = 128
NC, NS = 2, 16
NW = NC * NS
E2 = 2 * 320000
E2P = 655360
EPW = E2P // NW
RPZ = NP // NS

CHD = 128
NCHD = EPW // CHD
IBD = 32

CH = 64
NCH = EPW // CH
IB = 32
NBLK = NCH // IB
NBUF = 4

_mesh = plsc.VectorSubcoreMesh(core_axis_name="c", subcore_axis_name="s")


def _sc_degree(d2, zerosW, onesW):
    @functools.partial(
        pl.kernel,
        out_type=jax.ShapeDtypeStruct((NC, NP, DEGW), jnp.float32),
        mesh=_mesh,
        scratch_types=[
            pltpu.VMEM((IBD, CHD), jnp.int32),
            pltpu.VMEM((CHD, DEGW), jnp.float32),
            pltpu.VMEM_SHARED((NP, DEGW), jnp.float32),
            pltpu.SemaphoreType.DMA,
        ],
    )
    def k(d_hbm, z_hbm, o_hbm, out_hbm, didx, ones, acc, sem):
        cid = lax.axis_index("c")
        sid = lax.axis_index("s")
        wid = sid * NC + cid
        rz = NP // NS
        pltpu.sync_copy(z_hbm.at[pl.ds(sid * rz, rz)],
                        acc.at[pl.ds(sid * rz, rz)])
        pltpu.sync_copy(o_hbm, ones)
        plsc.subcore_barrier()

        @pl.loop(0, NCHD // IBD)
        def _(blk):
            pltpu.sync_copy(d_hbm.at[pl.ds(wid * NCHD + blk * IBD, IBD)], didx)

            @pl.loop(0, IBD)
            def _(c):
                pltpu.async_copy(ones, acc.at[didx.at[c]], sem, add=True)

            @pl.loop(0, IBD)
            def _(c):
                pltpu.make_async_copy(ones, acc.at[didx.at[0]], sem).wait()

        plsc.subcore_barrier()
        pltpu.sync_copy(acc.at[pl.ds(sid * rz, rz)],
                        out_hbm.at[cid, pl.ds(sid * rz, rz)])

    return k(d2, zerosW, onesW)


def _sc_scatter(m, s2, d2, zerosD):
    @functools.partial(
        pl.kernel,
        out_type=jax.ShapeDtypeStruct((NC, NP, D), jnp.float32),
        mesh=_mesh,
        scratch_types=[
            pltpu.VMEM((IB, CH), jnp.int32),
            pltpu.VMEM((IB, CH), jnp.int32),
            pltpu.VMEM((IB, CH), jnp.int32),
            pltpu.VMEM((IB, CH), jnp.int32),
            pltpu.VMEM((CH, D), jnp.float32),
            pltpu.VMEM((CH, D), jnp.float32),
            pltpu.VMEM((CH, D), jnp.float32),
            pltpu.VMEM((CH, D), jnp.float32),
            pltpu.VMEM_SHARED((NP, D), jnp.float32),
            pltpu.SemaphoreType.DMA,
            pltpu.SemaphoreType.DMA,
            pltpu.SemaphoreType.DMA,
            pltpu.SemaphoreType.DMA,
            pltpu.SemaphoreType.DMA,
            pltpu.SemaphoreType.DMA,
            pltpu.SemaphoreType.DMA,
            pltpu.SemaphoreType.DMA,
            pltpu.SemaphoreType.DMA,
            pltpu.SemaphoreType.DMA,
        ],
    )
    def k(m_hbm, s_hbm, d_hbm, z_hbm, out_hbm,
          si0, si1, di0, di1, b0, b1, b2, b3, acc,
          g0, g1, g2, g3, t0, t1, t2, t3, ip0, ip1):
        cid = lax.axis_index("c")
        sid = lax.axis_index("s")
        wid = sid * NC + cid
        bufs = (b0, b1, b2, b3)
        gsem = (g0, g1, g2, g3)
        ssem = (t0, t1, t2, t3)
        sidx = (si0, si1)
        didx = (di0, di1)
        isem = (ip0, ip1)

        pltpu.sync_copy(z_hbm.at[pl.ds(sid * RPZ, RPZ)],
                        acc.at[pl.ds(sid * RPZ, RPZ)])
        base0 = wid * NCH
        pltpu.async_copy(s_hbm.at[pl.ds(base0, IB)], sidx[0], isem[0])
        pltpu.async_copy(d_hbm.at[pl.ds(base0, IB)], didx[0], isem[0])
        plsc.subcore_barrier()

        def gather(ix, c, buf, sem):
            pltpu.async_copy(m_hbm.at[ix.at[c]], buf, sem)

        def gwait(buf, sem):
            pltpu.make_async_copy(m_hbm.at[sidx[0].at[0]], buf, sem).wait()

        def scat(ix, c, buf, sem):
            pltpu.async_copy(buf, acc.at[ix.at[c]], sem, add=True)

        def swait(buf, sem):
            pltpu.make_async_copy(buf, acc.at[didx[0].at[0]], sem).wait()

        for blk in range(NBLK):
            P = blk & 1
            pltpu.make_async_copy(s_hbm.at[pl.ds(0, IB)], sidx[P],
                                  isem[P]).wait()
            pltpu.make_async_copy(d_hbm.at[pl.ds(0, IB)], didx[P],
                                  isem[P]).wait()
            if blk + 1 < NBLK:
                nbase = wid * NCH + (blk + 1) * IB
                pltpu.async_copy(s_hbm.at[pl.ds(nbase, IB)], sidx[1 - P],
                                 isem[1 - P])
                pltpu.async_copy(d_hbm.at[pl.ds(nbase, IB)], didx[1 - P],
                                 isem[1 - P])
            for t in range(NBUF):
                if blk > 0:
                    swait(bufs[t], ssem[t])
                gather(sidx[P], t, bufs[t], gsem[t])

            @pl.loop(0, IB // NBUF - 1)
            def _(j):
                c = NBUF * j
                for t in range(NBUF):
                    gwait(bufs[t], gsem[t])
                    scat(didx[P], c + t, bufs[t], ssem[t])
                for t in range(NBUF):
                    swait(bufs[t], ssem[t])
                    gather(sidx[P], c + NBUF + t, bufs[t], gsem[t])

            ce = IB - NBUF
            for t in range(NBUF):
                gwait(bufs[t], gsem[t])
                scat(didx[P], ce + t, bufs[t], ssem[t])

        for t in range(NBUF):
            swait(bufs[t], ssem[t])
        plsc.subcore_barrier()
        pltpu.sync_copy(acc.at[pl.ds(sid * RPZ, RPZ)],
                        out_hbm.at[cid, pl.ds(sid * RPZ, RPZ)])

    return k(m, s2, d2, zerosD)


def _tc_norm_m(degp, h_pad):
    BR = 512

    def body(deg_ref, h_ref, norm_ref, m_ref):
        p = deg_ref[...]
        deg = p[0, :, 0:1] + p[1, :, 0:1] + 1.0
        nb = jnp.broadcast_to(lax.rsqrt(deg), (BR, D))
        norm_ref[...] = nb
        m_ref[...] = h_ref[...] * nb

    return pl.pallas_call(
        body,
        grid=(NP // BR,),
        in_specs=[
            pl.BlockSpec((NC, BR, DEGW), lambda i: (0, i, 0)),
            pl.BlockSpec((BR, D), lambda i: (i, 0)),
        ],
        out_specs=[
            pl.BlockSpec((BR, D), lambda i: (i, 0)),
            pl.BlockSpec((BR, D), lambda i: (i, 0)),
        ],
        out_shape=[
            jax.ShapeDtypeStruct((NP, D), jnp.float32),
            jax.ShapeDtypeStruct((NP, D), jnp.float32),
        ],
    )(degp, h_pad)


def _tc_layer(p, m, normb, W, b2d, last):
    rows = N_NODES if last else NP
    BR = 1000 if last else 512

    def body(p_ref, m_ref, n_ref, w_ref, b_ref, o_ref):
        pp = p_ref[...]
        agg = (pp[0] + pp[1] + m_ref[...]) * n_ref[...]
        y = jnp.dot(agg, w_ref[...], preferred_element_type=jnp.float32)
        y = y + b_ref[...]
        if not last:
            y = y * n_ref[...]
        o_ref[...] = y

    return pl.pallas_call(
        body,
        grid=(rows // BR,),
        in_specs=[
            pl.BlockSpec((NC, BR, D), lambda i: (0, i, 0)),
            pl.BlockSpec((BR, D), lambda i: (i, 0)),
            pl.BlockSpec((BR, D), lambda i: (i, 0)),
            pl.BlockSpec((D, D), lambda i: (0, 0)),
            pl.BlockSpec((1, D), lambda i: (0, 0)),
        ],
        out_specs=pl.BlockSpec((BR, D), lambda i: (i, 0)),
        out_shape=jax.ShapeDtypeStruct((rows, D), jnp.float32),
    )(p, m, normb, W, b2d)


def kernel(h, src, dst, W0, b0, W1, b1, W2, b2):
    pad = E2P - E2
    pad_idx = (jnp.arange(pad, dtype=jnp.int32) % (NP - N_NODES)) + N_NODES
    s_all = jnp.concatenate([src, dst, pad_idx])
    d_all = jnp.concatenate([dst, src, pad_idx])
    s2 = s_all.reshape(E2P // CH, CH)
    d2 = d_all.reshape(E2P // CH, CH)
    d2w = d_all.reshape(E2P // CHD, CHD)
    h_pad = jnp.pad(h, ((0, NP - N_NODES), (0, 0)))
    zerosD = jnp.zeros((NP, D), jnp.float32)
    zerosW = jnp.zeros((NP, DEGW), jnp.float32)
    onesW = jnp.ones((CHD, DEGW), jnp.float32)

    degp = _sc_degree(d2w, zerosW, onesW)
    normb, m = _tc_norm_m(degp, h_pad)
    out = None
    for i, (W, b) in enumerate(((W0, b0), (W1, b1), (W2, b2))):
        p = _sc_scatter(m, s2, d2, zerosD)
        out = _tc_layer(p, m, normb, W, b.reshape(1, D), last=(i == 2))
        m = out
    return out


# speedup vs baseline: 18.1324x; 1.1455x over previous
import functools

import jax
import jax.numpy as jnp
from jax import lax
from jax.experimental import pallas as pl
from jax.experimental.pallas import tpu as pltpu
from jax.experimental.pallas import tpu_sc as plsc

N_NODES = 10000
NP = 10240
D = 128
DEGW = 128
NC, NS = 2, 16
NW = NC * NS
E2 = 2 * 320000
E2P = 655360
EPW = E2P // NW
RPZ = NP // NS

CHD = 128
NCHD = EPW // CHD
IBD = 32

CH = 64
NCH = EPW // CH
IB = 32
NBLK = NCH // IB
NBUF = 4

_mesh = plsc.VectorSubcoreMesh(core_axis_name="c", subcore_axis_name="s")


def _sc_degree(d2, zeros1, ones1):
    @functools.partial(
        pl.kernel,
        out_type=jax.ShapeDtypeStruct((NC, NP), jnp.float32),
        mesh=_mesh,
        scratch_types=[
            pltpu.VMEM((IBD, CHD), jnp.int32),
            pltpu.VMEM((CHD,), jnp.float32),
            pltpu.VMEM_SHARED((NP,), jnp.float32),
            pltpu.SemaphoreType.DMA,
        ],
    )
    def k(d_hbm, z_hbm, o_hbm, out_hbm, didx, ones, acc, sem):
        cid = lax.axis_index("c")
        sid = lax.axis_index("s")
        wid = sid * NC + cid
        rz = NP // NS
        pltpu.sync_copy(z_hbm.at[pl.ds(sid * rz, rz)],
                        acc.at[pl.ds(sid * rz, rz)])
        pltpu.sync_copy(o_hbm, ones)
        plsc.subcore_barrier()

        @pl.loop(0, NCHD // IBD)
        def _(blk):
            pltpu.sync_copy(d_hbm.at[pl.ds(wid * NCHD + blk * IBD, IBD)], didx)

            @pl.loop(0, IBD)
            def _(c):
                pltpu.async_copy(ones, acc.at[didx.at[c]], sem, add=True)

            @pl.loop(0, IBD)
            def _(c):
                pltpu.make_async_copy(ones, acc.at[didx.at[0]], sem).wait()

        plsc.subcore_barrier()
        pltpu.sync_copy(acc.at[pl.ds(sid * rz, rz)],
                        out_hbm.at[cid, pl.ds(sid * rz, rz)])

    return k(d2, zeros1, ones1)


def _sc_scatter(m, s2, d2, zerosD):
    @functools.partial(
        pl.kernel,
        out_type=jax.ShapeDtypeStruct((NC, NP, D), jnp.float32),
        mesh=_mesh,
        scratch_types=[
            pltpu.VMEM((IB, CH), jnp.int32),
            pltpu.VMEM((IB, CH), jnp.int32),
            pltpu.VMEM((IB, CH), jnp.int32),
            pltpu.VMEM((IB, CH), jnp.int32),
            pltpu.VMEM((CH, D), jnp.float32),
            pltpu.VMEM((CH, D), jnp.float32),
            pltpu.VMEM((CH, D), jnp.float32),
            pltpu.VMEM((CH, D), jnp.float32),
            pltpu.VMEM_SHARED((NP, D), jnp.float32),
            pltpu.SemaphoreType.DMA,
            pltpu.SemaphoreType.DMA,
            pltpu.SemaphoreType.DMA,
            pltpu.SemaphoreType.DMA,
            pltpu.SemaphoreType.DMA,
            pltpu.SemaphoreType.DMA,
            pltpu.SemaphoreType.DMA,
            pltpu.SemaphoreType.DMA,
            pltpu.SemaphoreType.DMA,
            pltpu.SemaphoreType.DMA,
        ],
    )
    def k(m_hbm, s_hbm, d_hbm, z_hbm, out_hbm,
          si0, si1, di0, di1, b0, b1, b2, b3, acc,
          g0, g1, g2, g3, t0, t1, t2, t3, ip0, ip1):
        cid = lax.axis_index("c")
        sid = lax.axis_index("s")
        wid = sid * NC + cid
        bufs = (b0, b1, b2, b3)
        gsem = (g0, g1, g2, g3)
        ssem = (t0, t1, t2, t3)
        sidx = (si0, si1)
        didx = (di0, di1)
        isem = (ip0, ip1)

        pltpu.sync_copy(z_hbm.at[pl.ds(sid * RPZ, RPZ)],
                        acc.at[pl.ds(sid * RPZ, RPZ)])
        base0 = wid * NCH
        pltpu.async_copy(s_hbm.at[pl.ds(base0, IB)], sidx[0], isem[0])
        pltpu.async_copy(d_hbm.at[pl.ds(base0, IB)], didx[0], isem[0])
        plsc.subcore_barrier()

        def gather(ix, c, buf, sem):
            pltpu.async_copy(m_hbm.at[ix.at[c]], buf, sem)

        def gwait(buf, sem):
            pltpu.make_async_copy(m_hbm.at[sidx[0].at[0]], buf, sem).wait()

        def scat(ix, c, buf, sem):
            pltpu.async_copy(buf, acc.at[ix.at[c]], sem, add=True)

        def swait(buf, sem):
            pltpu.make_async_copy(buf, acc.at[didx[0].at[0]], sem).wait()

        for blk in range(NBLK):
            P = blk & 1
            pltpu.make_async_copy(s_hbm.at[pl.ds(0, IB)], sidx[P],
                                  isem[P]).wait()
            pltpu.make_async_copy(d_hbm.at[pl.ds(0, IB)], didx[P],
                                  isem[P]).wait()
            if blk + 1 < NBLK:
                nbase = wid * NCH + (blk + 1) * IB
                pltpu.async_copy(s_hbm.at[pl.ds(nbase, IB)], sidx[1 - P],
                                 isem[1 - P])
                pltpu.async_copy(d_hbm.at[pl.ds(nbase, IB)], didx[1 - P],
                                 isem[1 - P])
            for t in range(NBUF):
                if blk > 0:
                    swait(bufs[t], ssem[t])
                gather(sidx[P], t, bufs[t], gsem[t])

            @pl.loop(0, IB // NBUF - 1)
            def _(j):
                c = NBUF * j
                for t in range(NBUF):
                    gwait(bufs[t], gsem[t])
                    scat(didx[P], c + t, bufs[t], ssem[t])
                for t in range(NBUF):
                    swait(bufs[t], ssem[t])
                    gather(sidx[P], c + NBUF + t, bufs[t], gsem[t])

            ce = IB - NBUF
            for t in range(NBUF):
                gwait(bufs[t], gsem[t])
                scat(didx[P], ce + t, bufs[t], ssem[t])

        for t in range(NBUF):
            swait(bufs[t], ssem[t])
        plsc.subcore_barrier()
        pltpu.sync_copy(acc.at[pl.ds(sid * RPZ, RPZ)],
                        out_hbm.at[cid, pl.ds(sid * RPZ, RPZ)])

    return k(m, s2, d2, zerosD)


def _tc_norm_m(degp, h_pad):
    BR = 512

    def body(deg_ref, h_ref, norm_ref, m_ref):
        p = deg_ref[...]
        deg = p[:, 0:1] + p[:, 1:2] + 1.0
        nb = jnp.broadcast_to(lax.rsqrt(deg), (BR, D))
        norm_ref[...] = nb
        m_ref[...] = h_ref[...] * nb

    return pl.pallas_call(
        body,
        grid=(NP // BR,),
        in_specs=[
            pl.BlockSpec((BR, NC), lambda i: (i, 0)),
            pl.BlockSpec((BR, D), lambda i: (i, 0)),
        ],
        out_specs=[
            pl.BlockSpec((BR, D), lambda i: (i, 0)),
            pl.BlockSpec((BR, D), lambda i: (i, 0)),
        ],
        out_shape=[
            jax.ShapeDtypeStruct((NP, D), jnp.float32),
            jax.ShapeDtypeStruct((NP, D), jnp.float32),
        ],
    )(degp, h_pad)


def _tc_layer(p, m, normb, W, b2d, last):
    rows = N_NODES if last else NP
    BR = 1000 if last else 512

    def body(p_ref, m_ref, n_ref, w_ref, b_ref, o_ref):
        pp = p_ref[...]
        agg = (pp[0] + pp[1] + m_ref[...]) * n_ref[...]
        y = jnp.dot(agg, w_ref[...], preferred_element_type=jnp.float32)
        y = y + b_ref[...]
        if not last:
            y = y * n_ref[...]
        o_ref[...] = y

    return pl.pallas_call(
        body,
        grid=(rows // BR,),
        in_specs=[
            pl.BlockSpec((NC, BR, D), lambda i: (0, i, 0)),
            pl.BlockSpec((BR, D), lambda i: (i, 0)),
            pl.BlockSpec((BR, D), lambda i: (i, 0)),
            pl.BlockSpec((D, D), lambda i: (0, 0)),
            pl.BlockSpec((1, D), lambda i: (0, 0)),
        ],
        out_specs=pl.BlockSpec((BR, D), lambda i: (i, 0)),
        out_shape=jax.ShapeDtypeStruct((rows, D), jnp.float32),
    )(p, m, normb, W, b2d)


def kernel(h, src, dst, W0, b0, W1, b1, W2, b2):
    pad = E2P - E2
    pad_idx = (jnp.arange(pad, dtype=jnp.int32) % (NP - N_NODES)) + N_NODES
    s_all = jnp.concatenate([src, dst, pad_idx])
    d_all = jnp.concatenate([dst, src, pad_idx])
    s2 = s_all.reshape(E2P // CH, CH)
    d2 = d_all.reshape(E2P // CH, CH)
    d2w = d_all.reshape(E2P // CHD, CHD)
    h_pad = jnp.pad(h, ((0, NP - N_NODES), (0, 0)))
    zerosD = jnp.zeros((NP, D), jnp.float32)
    zeros1 = jnp.zeros((NP,), jnp.float32)
    ones1 = jnp.ones((CHD,), jnp.float32)

    degp = _sc_degree(d2w, zeros1, ones1)
    degt = degp.T
    normb, m = _tc_norm_m(degt, h_pad)
    out = None
    for i, (W, b) in enumerate(((W0, b0), (W1, b1), (W2, b2))):
        p = _sc_scatter(m, s2, d2, zerosD)
        out = _tc_layer(p, m, normb, W, b.reshape(1, D), last=(i == 2))
        m = out
    return out
